# Initial kernel scaffold; baseline (speedup 1.0000x reference)
import functools

import jax
import jax.numpy as jnp
from jax import lax
from jax.experimental import pallas as pl
from jax.experimental.pallas import tpu as pltpu
from jax.experimental.pallas import tpu_sc as plsc

N = 10000
DEG = 32
E = N * DEG
U = 128

NB = 8
EB = NB * DEG
NBLK = N // NB
NW = 32
KMAX = (NBLK + NW - 1) // NW
LANES = 16


def _tc_body(ns_ref, w_ref, kab_ref, x_ref, a_ref):
    x = jnp.dot(ns_ref[...], w_ref[...], preferred_element_type=jnp.float32)
    x_ref[...] = x
    a_ref[...] = jnp.dot(x, kab_ref[...], preferred_element_type=jnp.float32)


def _dense(ns2, w, kab):
    rows = 1000
    return pl.pallas_call(
        _tc_body,
        grid=(N // rows,),
        in_specs=[
            pl.BlockSpec((rows, U), lambda i: (i, 0)),
            pl.BlockSpec((U, U), lambda i: (0, 0)),
            pl.BlockSpec((U, 2), lambda i: (0, 0)),
        ],
        out_specs=[
            pl.BlockSpec((rows, U), lambda i: (i, 0)),
            pl.BlockSpec((rows, 2), lambda i: (i, 0)),
        ],
        out_shape=[
            jax.ShapeDtypeStruct((N, U), jnp.float32),
            jax.ShapeDtypeStruct((N, 2), jnp.float32),
        ],
    )(ns2, w, kab)


def _sc_body(x_hbm, a2_hbm, src2_hbm, src1_hbm, out_hbm,
             idx2_v, idx1_v, rows_v, a2_v, scores_v, outb_v, sem):
    c = lax.axis_index("c")
    s = lax.axis_index("s")
    w = s * 2 + c

    pltpu.sync_copy(a2_hbm, a2_v)

    zi = jnp.zeros((LANES,), jnp.int32)

    def do_block(b):
        pltpu.sync_copy(src2_hbm.at[pl.ds(2 * b, 2)], idx2_v)
        pltpu.sync_copy(src1_hbm.at[pl.ds(EB * b, EB)], idx1_v)
        d0 = pltpu.async_copy(x_hbm.at[idx2_v.at[0]], rows_v.at[pl.ds(0, 128)], sem)
        d1 = pltpu.async_copy(x_hbm.at[idx2_v.at[1]], rows_v.at[pl.ds(128, 128)], sem)
        d0.wait()
        d1.wait()

        nbase = NB * b

        def node_body(i, _):
            eb = DEG * i
            b_n = a2_v[nbase + i, 1]
            ssum_vec = jnp.zeros((LANES,), jnp.float32)
            for j in range(DEG // LANES):
                idxc = idx1_v[pl.ds(eb + LANES * j, LANES)]
                u = plsc.load_gather(a2_v, [idxc, zi])
                t = u + b_n
                t = jnp.where(t >= 0.0, t, 0.2 * t)
                t = jnp.clip(t, -2.0, 2.0)
                sc = jnp.exp(t)
                scores_v[pl.ds(eb + LANES * j, LANES)] = sc
                ssum_vec = ssum_vec + sc
            ssum = jnp.sum(ssum_vec)
            inv = 1.0 / jnp.broadcast_to(ssum, (LANES,))
            for cidx in range(U // LANES):
                acc = jnp.zeros((LANES,), jnp.float32)
                for e in range(DEG):
                    wsc = scores_v[eb + e]
                    acc = acc + wsc * rows_v[eb + e, pl.ds(LANES * cidx, LANES)]
                outb_v[i, pl.ds(LANES * cidx, LANES)] = acc * inv
            return 0

        lax.fori_loop(0, NB, node_body, 0)
        pltpu.sync_copy(outb_v, out_hbm.at[pl.ds(nbase, NB)])

    def k_body(k, _):
        b = w + NW * k

        @pl.when(b < NBLK)
        def _():
            do_block(b)

        return 0

    lax.fori_loop(0, KMAX, k_body, 0)


def _sparse(x, a2, src2, src1):
    mesh = plsc.VectorSubcoreMesh(core_axis_name="c", subcore_axis_name="s")
    return pl.kernel(
        _sc_body,
        out_type=jax.ShapeDtypeStruct((N, U), jnp.float32),
        mesh=mesh,
        scratch_types=[
            pltpu.VMEM((2, 128), jnp.int32),
            pltpu.VMEM((EB,), jnp.int32),
            pltpu.VMEM((EB, U), jnp.float32),
            pltpu.VMEM((N, 2), jnp.float32),
            pltpu.VMEM((EB,), jnp.float32),
            pltpu.VMEM((NB, U), jnp.float32),
            pltpu.SemaphoreType.DMA,
        ],
    )(x, a2, src2, src1)


def kernel(node_states, edges, kernel, kernel_attention):
    ns2 = node_states[0]
    kab = jnp.stack(
        [kernel_attention[:U, 0], kernel_attention[U:, 0]], axis=1
    )
    x, a2 = _dense(ns2, kernel, kab)
    src1 = edges[:, 0]
    src2 = src1.reshape(E // 128, 128)
    out = _sparse(x, a2, src2, src1)
    return out[None]


# speedup vs baseline: 16.8728x; 16.8728x over previous
import functools

import jax
import jax.numpy as jnp
from jax import lax
from jax.experimental import pallas as pl
from jax.experimental.pallas import tpu as pltpu
from jax.experimental.pallas import tpu_sc as plsc

N = 10000
DEG = 32
E = N * DEG
U = 128

NB = 8
EB = NB * DEG
NBLK = N // NB
NW = 32
KMAX = (NBLK + NW - 1) // NW
LANES = 16


def _tc_body(ns_ref, w_ref, kab_ref, x_ref, a_ref):
    x = jnp.dot(ns_ref[...], w_ref[...], preferred_element_type=jnp.float32)
    x_ref[...] = x
    a_ref[...] = jnp.dot(x, kab_ref[...], preferred_element_type=jnp.float32)


def _dense(ns2, w, kab):
    rows = 1000
    return pl.pallas_call(
        _tc_body,
        grid=(N // rows,),
        in_specs=[
            pl.BlockSpec((rows, U), lambda i: (i, 0)),
            pl.BlockSpec((U, U), lambda i: (0, 0)),
            pl.BlockSpec((U, 2), lambda i: (0, 0)),
        ],
        out_specs=[
            pl.BlockSpec((rows, U), lambda i: (i, 0)),
            pl.BlockSpec((rows, 2), lambda i: (i, 0)),
        ],
        out_shape=[
            jax.ShapeDtypeStruct((N, U), jnp.float32),
            jax.ShapeDtypeStruct((N, 2), jnp.float32),
        ],
    )(ns2, w, kab)


def _sc_body(x_hbm, asrc_hbm, adst_hbm, src2_hbm, out_hbm,
             idx2_v, rows_v, asrc_v, adst_v, outb_v, sem):
    c = lax.axis_index("c")
    s = lax.axis_index("s")
    w = s * 2 + c

    pltpu.sync_copy(asrc_hbm, asrc_v)
    pltpu.sync_copy(adst_hbm, adst_v.at[pl.ds(0, N)])

    def do_block(b):
        pltpu.sync_copy(src2_hbm.at[pl.ds(2 * b, 2)], idx2_v)
        d0 = pltpu.async_copy(x_hbm.at[idx2_v.at[0]], rows_v.at[pl.ds(0, 128)], sem)
        d1 = pltpu.async_copy(x_hbm.at[idx2_v.at[1]], rows_v.at[pl.ds(128, 128)], sem)
        d0.wait()
        d1.wait()

        nbase = NB * b
        ad = adst_v[pl.ds(nbase, LANES)]

        for i in range(NB):
            eb = DEG * i
            b_n = ad[i]
            wvecs = []
            ssum_vec = jnp.zeros((LANES,), jnp.float32)
            for j in range(DEG // LANES):
                q = 2 * i + j
                idxc = idx2_v[q // 8, pl.ds(LANES * (q % 8), LANES)]
                u = plsc.load_gather(asrc_v, [idxc])
                t = u + b_n
                t = jnp.where(t >= 0.0, t, 0.2 * t)
                t = jnp.clip(t, -2.0, 2.0)
                sc = jnp.exp(t)
                wvecs.append(sc)
                ssum_vec = ssum_vec + sc
            inv = 1.0 / jnp.broadcast_to(jnp.sum(ssum_vec), (LANES,))
            wvecs = [wv * inv for wv in wvecs]
            for cidx in range(U // LANES):
                acc = jnp.zeros((LANES,), jnp.float32)
                for e in range(DEG):
                    wsc = wvecs[e // LANES][e % LANES]
                    acc = acc + wsc * rows_v[eb + e, pl.ds(LANES * cidx, LANES)]
                outb_v[i, pl.ds(LANES * cidx, LANES)] = acc
        pltpu.sync_copy(outb_v, out_hbm.at[pl.ds(nbase, NB)])

    def k_body(k, _):
        b = w + NW * k

        @pl.when(b < NBLK)
        def _():
            do_block(b)

        return 0

    lax.fori_loop(0, KMAX, k_body, 0)


def _sparse(x, asrc, adst, src2):
    mesh = plsc.VectorSubcoreMesh(core_axis_name="c", subcore_axis_name="s")
    return pl.kernel(
        _sc_body,
        out_type=jax.ShapeDtypeStruct((N, U), jnp.float32),
        mesh=mesh,
        compiler_params=pltpu.CompilerParams(needs_layout_passes=False),
        scratch_types=[
            pltpu.VMEM((2, 128), jnp.int32),
            pltpu.VMEM((EB, U), jnp.float32),
            pltpu.VMEM((N,), jnp.float32),
            pltpu.VMEM((N + LANES,), jnp.float32),
            pltpu.VMEM((NB, U), jnp.float32),
            pltpu.SemaphoreType.DMA,
        ],
    )(x, asrc, adst, src2)


def kernel(node_states, edges, kernel, kernel_attention):
    ns2 = node_states[0]
    kab = jnp.stack(
        [kernel_attention[:U, 0], kernel_attention[U:, 0]], axis=1
    )
    x, a2 = _dense(ns2, kernel, kab)
    src2 = edges[:, 0].reshape(E // 128, 128)
    out = _sparse(x, a2[:, 0], a2[:, 1], src2)
    return out[None]
